# baseline (speedup 1.0000x reference)
import functools

import numpy as np
import jax
import jax.numpy as jnp
from jax.experimental import pallas as pl
from jax.experimental.pallas import tpu as pltpu

_DIM_IN = 32
_DIM = 64
_HID = 256
_TN = 2000


def _penc_consts():
    ch = _DIM_IN
    exps = np.arange(0, ch, 2, dtype=np.float32) / np.float32(ch)
    inv_freq = (1.0 / (np.float32(10000.0) ** exps)).astype(np.float32)
    freq_row = np.repeat(inv_freq, 2).reshape(1, ch)
    sin_mask = (np.arange(ch) % 2 == 0).reshape(1, ch)
    return freq_row, sin_mask


_FREQ_ROW, _SIN_MASK = _penc_consts()


def _ss_kernel(t_ref, wss_ref, bss_ref, ss_ref):
    tt = t_ref[...]
    st = tt * jax.lax.logistic(tt)
    ss_ref[...] = (
        jax.lax.dot_general(st, wss_ref[...], (((1,), (0,)), ((), ())),
                            preferred_element_type=jnp.float32)
        + bss_ref[...]
    )


def _ffn_kernel(x_ref, ss_ref, wx_ref, wp_ref, wo_ref, freq_ref, mask_ref,
                o_ref, *, tn):
    j = pl.program_id(1)
    x = x_ref[0]

    rows = jax.lax.broadcasted_iota(jnp.int32, (tn, _DIM_IN), 0)
    pos = (rows + j * tn).astype(jnp.float32)
    angle = pos * freq_ref[...]
    mask = mask_ref[...]
    penc = mask * jnp.sin(angle) + (1.0 - mask) * jnp.cos(angle)

    nrm2 = jnp.sum(x * x, axis=1, keepdims=True) + jnp.float32(_DIM_IN / 2)
    inv = jnp.float32(np.sqrt(_DIM)) * jax.lax.rsqrt(nrm2)

    h = jax.lax.dot_general(x, wx_ref[...], (((1,), (0,)), ((), ())),
                            preferred_element_type=jnp.float32)
    h = h + jax.lax.dot_general(penc, wp_ref[...], (((1,), (0,)), ((), ())),
                                preferred_element_type=jnp.float32)
    h = h * inv
    h = h * jax.lax.logistic(h)

    ss = ss_ref[0]
    h = h * (ss[:, :_HID] + 1.0) + ss[:, _HID:]

    o_ref[0] = jax.lax.dot_general(h, wo_ref[...], (((1,), (0,)), ((), ())),
                                   preferred_element_type=jnp.float32)


def kernel(x, t, w_in, w_out, w_ss, b_ss):
    b, n, c = x.shape
    hid2 = w_ss.shape[1]
    ss = pl.pallas_call(
        _ss_kernel,
        out_shape=jax.ShapeDtypeStruct((b, hid2), jnp.float32),
    )(t, w_ss, b_ss.reshape(1, hid2))
    ss3 = ss.reshape(b, 1, hid2)

    wx = w_in[:c]
    wp = w_in[c:]
    nb = n // _TN
    out = pl.pallas_call(
        functools.partial(_ffn_kernel, tn=_TN),
        grid=(b, nb),
        in_specs=[
            pl.BlockSpec((1, _TN, c), lambda i, j: (i, j, 0)),
            pl.BlockSpec((1, 1, hid2), lambda i, j: (i, 0, 0)),
            pl.BlockSpec((c, _HID), lambda i, j: (0, 0)),
            pl.BlockSpec((c, _HID), lambda i, j: (0, 0)),
            pl.BlockSpec((_HID, _DIM), lambda i, j: (0, 0)),
            pl.BlockSpec((1, c), lambda i, j: (0, 0)),
            pl.BlockSpec((1, c), lambda i, j: (0, 0)),
        ],
        out_specs=pl.BlockSpec((1, _TN, _DIM), lambda i, j: (i, j, 0)),
        out_shape=jax.ShapeDtypeStruct((b, n, _DIM), jnp.float32),
        compiler_params=pltpu.CompilerParams(
            dimension_semantics=("parallel", "arbitrary")),
    )(x, ss3, wx, wp, w_out,
      jnp.asarray(_FREQ_ROW), jnp.asarray(_SIN_MASK, dtype=jnp.float32))
    return out


# speedup vs baseline: 2.1650x; 2.1650x over previous
import functools

import numpy as np
import jax
import jax.numpy as jnp
from jax.experimental import pallas as pl
from jax.experimental.pallas import tpu as pltpu

_DIM_IN = 32
_DIM = 64
_HID = 256
_TN = 2000


def _penc_consts(tn):
    ch = _DIM_IN
    exps = np.arange(0, ch, 2, dtype=np.float32) / np.float32(ch)
    inv_freq = (1.0 / (np.float32(10000.0) ** exps)).astype(np.float64)
    freq_row = np.repeat(inv_freq, 2).reshape(1, ch)
    sin_mask = ((np.arange(ch) % 2) == 0).reshape(1, ch)
    r = np.arange(tn, dtype=np.float64).reshape(tn, 1)
    ang = r * freq_row
    s_tab = np.sin(ang).astype(np.float32)
    c_tab = np.cos(ang).astype(np.float32)
    return (freq_row.astype(np.float32), sin_mask.astype(np.float32),
            s_tab, c_tab)


_FREQ_ROW, _SIN_MASK, _S_TAB, _C_TAB = _penc_consts(_TN)


def _ss_kernel(t_ref, wss_ref, bss_ref, ss_ref):
    tt = t_ref[...]
    st = tt * jax.lax.logistic(tt)
    ss_ref[...] = (
        jax.lax.dot_general(st, wss_ref[...], (((1,), (0,)), ((), ())),
                            preferred_element_type=jnp.float32)
        + bss_ref[...]
    )


def _ffn_kernel(x_ref, ss_ref, wx_ref, wp_ref, wo_ref, freq_ref, mask_ref,
                stab_ref, ctab_ref, o_ref, *, tn):
    j = pl.program_id(1)
    x = x_ref[0]

    base = (j * tn).astype(jnp.float32) * freq_ref[...]
    sb = jnp.sin(base)
    cb = jnp.cos(base)
    mask = mask_ref[...]
    a_row = mask * sb + (1.0 - mask) * cb
    b_row = mask * cb - (1.0 - mask) * sb
    penc = a_row * ctab_ref[...] + b_row * stab_ref[...]

    nrm2 = jnp.sum(x * x, axis=1, keepdims=True) + jnp.float32(_DIM_IN / 2)
    inv = jnp.float32(np.sqrt(_DIM)) * jax.lax.rsqrt(nrm2)

    h = jax.lax.dot_general(x * inv, wx_ref[...], (((1,), (0,)), ((), ())),
                            preferred_element_type=jnp.float32)
    h = h + jax.lax.dot_general(penc * inv, wp_ref[...], (((1,), (0,)), ((), ())),
                                preferred_element_type=jnp.float32)
    h = h * jax.lax.logistic(h)

    ss = ss_ref[0]
    h = h * (ss[:, :_HID] + 1.0) + ss[:, _HID:]

    o_ref[0] = jax.lax.dot_general(h, wo_ref[...], (((1,), (0,)), ((), ())),
                                   preferred_element_type=jnp.float32)


def kernel(x, t, w_in, w_out, w_ss, b_ss):
    b, n, c = x.shape
    hid2 = w_ss.shape[1]
    ss = pl.pallas_call(
        _ss_kernel,
        out_shape=jax.ShapeDtypeStruct((b, hid2), jnp.float32),
    )(t, w_ss, b_ss.reshape(1, hid2))
    ss3 = ss.reshape(b, 1, hid2)

    wx = w_in[:c]
    wp = w_in[c:]
    nb = n // _TN
    out = pl.pallas_call(
        functools.partial(_ffn_kernel, tn=_TN),
        grid=(b, nb),
        in_specs=[
            pl.BlockSpec((1, _TN, c), lambda i, j: (i, j, 0)),
            pl.BlockSpec((1, 1, hid2), lambda i, j: (i, 0, 0)),
            pl.BlockSpec((c, _HID), lambda i, j: (0, 0)),
            pl.BlockSpec((c, _HID), lambda i, j: (0, 0)),
            pl.BlockSpec((_HID, _DIM), lambda i, j: (0, 0)),
            pl.BlockSpec((1, c), lambda i, j: (0, 0)),
            pl.BlockSpec((1, c), lambda i, j: (0, 0)),
            pl.BlockSpec((_TN, c), lambda i, j: (0, 0)),
            pl.BlockSpec((_TN, c), lambda i, j: (0, 0)),
        ],
        out_specs=pl.BlockSpec((1, _TN, _DIM), lambda i, j: (i, j, 0)),
        out_shape=jax.ShapeDtypeStruct((b, n, _DIM), jnp.float32),
        compiler_params=pltpu.CompilerParams(
            dimension_semantics=("parallel", "arbitrary")),
    )(x, ss3, wx, wp, w_out,
      jnp.asarray(_FREQ_ROW), jnp.asarray(_SIN_MASK),
      jnp.asarray(_S_TAB), jnp.asarray(_C_TAB))
    return out


# speedup vs baseline: 5.5340x; 2.5561x over previous
import functools

import numpy as np
import jax
import jax.numpy as jnp
from jax.experimental import pallas as pl
from jax.experimental.pallas import tpu as pltpu

_DIM_IN = 32
_DIM = 64
_HID = 256
_TT = 6400


def _penc_consts(tt):
    ch = _DIM_IN
    exps = np.arange(0, ch, 2, dtype=np.float32) / np.float32(ch)
    inv_freq = (1.0 / (np.float32(10000.0) ** exps)).astype(np.float64)
    freq_col = np.repeat(inv_freq, 2).reshape(ch, 1)
    sin_mask = ((np.arange(ch) % 2) == 0).reshape(ch, 1)
    s = np.arange(tt, dtype=np.float64).reshape(1, tt)
    ang = freq_col * s
    s_tab = np.sin(ang).astype(np.float32)
    c_tab = np.cos(ang).astype(np.float32)
    ones = np.full((8, ch), 1.0 / _DIM, dtype=np.float32)
    return (freq_col.astype(np.float32), sin_mask.astype(np.float32),
            s_tab, c_tab, ones)


_FREQ_C, _MASK_C, _S_TAB, _C_TAB, _ONES = _penc_consts(_TT)


def _ss_kernel(t_ref, wss_ref, bss_ref, woT_ref, wobT_ref, biasT_ref):
    tt = t_ref[...]
    st = tt * jax.lax.logistic(tt)
    ss = (jax.lax.dot_general(st, wss_ref[...], (((1,), (0,)), ((), ())),
                              preferred_element_type=jnp.float32)
          + bss_ref[...])
    b = ss.shape[0]
    sc1 = ss[:, :_HID] + 1.0
    sh = ss[:, _HID:]
    woT = woT_ref[...]
    for i in range(b):
        row = jax.lax.slice(sc1, (i, 0), (i + 1, _HID))
        wobT_ref[i] = woT * jnp.broadcast_to(row, (_DIM, _HID))
    bias_all = jax.lax.dot_general(
        woT, sh, (((1,), (1,)), ((), ())),
        preferred_element_type=jnp.float32)
    for i in range(b):
        biasT_ref[i] = jax.lax.slice(bias_all, (0, i), (_DIM, i + 1))


def _ffn_kernel(x_ref, wxT_ref, wpT_ref, wobT_ref, biasT_ref, freq_ref,
                mask_ref, stab_ref, ctab_ref, ones_ref, o_ref, *, tt):
    j = pl.program_id(1)
    xt = x_ref[0]

    base = (j * tt).astype(jnp.float32) * freq_ref[...]
    sb = jnp.sin(base)
    cb = jnp.cos(base)
    m = mask_ref[...]
    a_col = m * sb + (1.0 - m) * cb
    b_col = m * cb - (1.0 - m) * sb
    penc = a_col * ctab_ref[...] + b_col * stab_ref[...]

    nrm2 = jax.lax.dot_general(ones_ref[...], (xt * xt).astype(jnp.bfloat16),
                               (((1,), (0,)), ((), ())),
                               preferred_element_type=jnp.float32)
    inv = jax.lax.rsqrt(nrm2 + jnp.float32(_DIM_IN / 2 / _DIM))
    inv = jax.lax.slice(inv, (0, 0), (1, inv.shape[1]))

    xn = (xt * inv).astype(jnp.bfloat16)
    pn = (penc * inv).astype(jnp.bfloat16)

    h = jax.lax.dot_general(wxT_ref[...], xn, (((1,), (0,)), ((), ())),
                            preferred_element_type=jnp.float32)
    h = h + jax.lax.dot_general(wpT_ref[...], pn, (((1,), (0,)), ((), ())),
                                preferred_element_type=jnp.float32)
    h = h * (0.5 + 0.5 * jnp.tanh(0.5 * h))

    o_ref[0] = (jax.lax.dot_general(wobT_ref[0], h.astype(jnp.bfloat16),
                                    (((1,), (0,)), ((), ())),
                                    preferred_element_type=jnp.float32)
                + biasT_ref[0])


def kernel(x, t, w_in, w_out, w_ss, b_ss):
    b, n, c = x.shape
    hid2 = w_ss.shape[1]
    wobT, biasT = pl.pallas_call(
        _ss_kernel,
        out_shape=(jax.ShapeDtypeStruct((b, _DIM, _HID), jnp.float32),
                   jax.ShapeDtypeStruct((b, _DIM, 1), jnp.float32)),
    )(t, w_ss, b_ss.reshape(1, hid2), w_out.T)

    xT = jnp.transpose(x, (0, 2, 1))
    nb = -(-n // _TT)
    outT = pl.pallas_call(
        functools.partial(_ffn_kernel, tt=_TT),
        grid=(b, nb),
        in_specs=[
            pl.BlockSpec((1, c, _TT), lambda i, j: (i, 0, j)),
            pl.BlockSpec((_HID, c), lambda i, j: (0, 0)),
            pl.BlockSpec((_HID, c), lambda i, j: (0, 0)),
            pl.BlockSpec((1, _DIM, _HID), lambda i, j: (i, 0, 0)),
            pl.BlockSpec((1, _DIM, 1), lambda i, j: (i, 0, 0)),
            pl.BlockSpec((c, 1), lambda i, j: (0, 0)),
            pl.BlockSpec((c, 1), lambda i, j: (0, 0)),
            pl.BlockSpec((c, _TT), lambda i, j: (0, 0)),
            pl.BlockSpec((c, _TT), lambda i, j: (0, 0)),
            pl.BlockSpec((8, c), lambda i, j: (0, 0)),
        ],
        out_specs=pl.BlockSpec((1, _DIM, _TT), lambda i, j: (i, 0, j)),
        out_shape=jax.ShapeDtypeStruct((b, _DIM, n), jnp.float32),
        compiler_params=pltpu.CompilerParams(
            dimension_semantics=("parallel", "arbitrary")),
    )(xT, w_in[:c].T.astype(jnp.bfloat16), w_in[c:].T.astype(jnp.bfloat16),
      wobT.astype(jnp.bfloat16), biasT,
      jnp.asarray(_FREQ_C), jnp.asarray(_MASK_C),
      jnp.asarray(_S_TAB), jnp.asarray(_C_TAB),
      jnp.asarray(_ONES).astype(jnp.bfloat16))
    return jnp.transpose(outT, (0, 2, 1))
